# baseline (device time: 94507 ns/iter reference)
import jax
import jax.numpy as jnp
from jax import lax
from jax.experimental import pallas as pl
from jax.experimental.pallas import tpu as pltpu

N_DEV = 4


def kernel(x, router_W, route_idx, expert_W, shared_W):
    n_tok, d_model = x.shape
    e_loc, _, d_ff = expert_W.shape
    n_exp = router_W.shape[1]

    def body(x_ref, rw_ref, idx_ref, ew_ref, sw_ref, out_ref,
             comm_ref, send_sems, recv_sems):
        my = lax.axis_index("i")
        left = lax.rem(my + N_DEV - 1, N_DEV)
        right = lax.rem(my + 1, N_DEV)

        barrier_sem = pltpu.get_barrier_semaphore()
        for nbr in [left, right]:
            pl.semaphore_signal(
                barrier_sem, inc=1,
                device_id=(nbr,), device_id_type=pl.DeviceIdType.MESH,
            )
        pl.semaphore_wait(barrier_sem, 2)

        xf = x_ref[:, :]
        scores = jnp.dot(xf, rw_ref[:, :], preferred_element_type=jnp.float32)
        probs = jax.nn.softmax(scores, axis=-1)
        idx = idx_ref[:, :]
        eye = lax.broadcasted_iota(jnp.int32, (n_tok, n_exp), 1)
        onehot = (idx == eye).astype(jnp.float32)
        p_sel = jnp.sum(probs * onehot, axis=-1, keepdims=True)

        xb = xf.astype(jnp.bfloat16)
        acc = jnp.zeros((n_tok, d_ff), jnp.float32)
        for e in range(e_loc):
            e_glob = my * e_loc + e
            coef = jnp.where(idx == e_glob, p_sel, 0.0)
            xs = (xf * coef).astype(jnp.bfloat16)
            acc = acc + jnp.dot(
                xs, ew_ref[e].astype(jnp.bfloat16),
                preferred_element_type=jnp.float32,
            )

        shared = jnp.dot(
            xb, sw_ref[:, :].astype(jnp.bfloat16),
            preferred_element_type=jnp.float32,
        )

        comm_ref[0, :, :] = acc.astype(jnp.bfloat16)
        for h in range(N_DEV - 1):
            rdma = pltpu.make_async_remote_copy(
                src_ref=comm_ref.at[h],
                dst_ref=comm_ref.at[h + 1],
                send_sem=send_sems.at[h],
                recv_sem=recv_sems.at[h],
                device_id=(right,),
                device_id_type=pl.DeviceIdType.MESH,
            )
            rdma.start()
            rdma.wait()

        total = acc + shared
        for s in range(1, N_DEV):
            total = total + comm_ref[s, :, :].astype(jnp.float32)
        out_ref[:, :] = total

    return pl.pallas_call(
        body,
        out_shape=jax.ShapeDtypeStruct((n_tok, d_ff), jnp.float32),
        in_specs=[pl.BlockSpec(memory_space=pltpu.VMEM)] * 5,
        out_specs=pl.BlockSpec(memory_space=pltpu.VMEM),
        scratch_shapes=[
            pltpu.VMEM((N_DEV, n_tok, d_ff), jnp.bfloat16),
            pltpu.SemaphoreType.DMA((N_DEV - 1,)),
            pltpu.SemaphoreType.DMA((N_DEV - 1,)),
        ],
        compiler_params=pltpu.CompilerParams(collective_id=0),
    )(x, router_W, route_idx, expert_W, shared_W)


# device time: 48649 ns/iter; 1.9426x vs baseline; 1.9426x over previous
import jax
import jax.numpy as jnp
from jax import lax
from jax.experimental import pallas as pl
from jax.experimental.pallas import tpu as pltpu

N_DEV = 4


def kernel(x, router_W, route_idx, expert_W, shared_W):
    n_tok, d_model = x.shape
    e_loc, _, d_ff = expert_W.shape
    n_exp = router_W.shape[1]
    half = n_tok // 2
    ch = half // N_DEV

    def body(x_ref, rw_ref, idx_ref, ew_ref, sw_ref, out_ref,
             part_ref,
             rs_send_r, rs_recv_r, rs_send_l, rs_recv_l,
             ag_r, ag_l,
             rs_ssem_r, rs_rsem_r, rs_ssem_l, rs_rsem_l,
             ag_ssem_r, ag_rsem_r, ag_ssem_l, ag_rsem_l):
        my = lax.axis_index("i")
        left = lax.rem(my + N_DEV - 1, N_DEV)
        right = lax.rem(my + 1, N_DEV)

        barrier_sem = pltpu.get_barrier_semaphore()
        for nbr in [left, right]:
            pl.semaphore_signal(
                barrier_sem, inc=1,
                device_id=(nbr,), device_id_type=pl.DeviceIdType.MESH,
            )
        pl.semaphore_wait(barrier_sem, 2)

        xf = x_ref[:, :]
        scores = jnp.dot(xf, rw_ref[:, :], preferred_element_type=jnp.float32)
        probs = jax.nn.softmax(scores, axis=-1)
        idx = idx_ref[:, :]
        eye = lax.broadcasted_iota(jnp.int32, (n_tok, n_exp), 1)
        onehot = (idx == eye).astype(jnp.float32)
        p_sel = jnp.sum(probs * onehot, axis=-1, keepdims=True)

        acc = jnp.zeros((n_tok, d_ff), jnp.float32)
        for e in range(e_loc):
            e_glob = my * e_loc + e
            coef = jnp.where(idx == e_glob, p_sel, 0.0)
            xs = (xf * coef).astype(jnp.bfloat16)
            acc = acc + jnp.dot(
                xs, ew_ref[e].astype(jnp.bfloat16),
                preferred_element_type=jnp.float32,
            )
        part_ref[:, :] = acc

        xb = xf.astype(jnp.bfloat16)
        out_ref[:, :] = jnp.dot(
            xb, sw_ref[:, :].astype(jnp.bfloat16),
            preferred_element_type=jnp.float32,
        )

        for t in range(N_DEV - 1):
            c_r = lax.rem(my - t + N_DEV, N_DEV)
            row_r = c_r * ch
            val_r = part_ref[pl.ds(row_r, ch), :]
            if t > 0:
                val_r = val_r + rs_recv_r[t - 1, :, :].astype(jnp.float32)
            rs_send_r[t, :, :] = val_r.astype(jnp.bfloat16)
            rdma_r = pltpu.make_async_remote_copy(
                src_ref=rs_send_r.at[t], dst_ref=rs_recv_r.at[t],
                send_sem=rs_ssem_r.at[t], recv_sem=rs_rsem_r.at[t],
                device_id=(right,), device_id_type=pl.DeviceIdType.MESH,
            )
            rdma_r.start()

            c_l = lax.rem(my + t, N_DEV)
            row_l = half + c_l * ch
            val_l = part_ref[pl.ds(row_l, ch), :]
            if t > 0:
                val_l = val_l + rs_recv_l[t - 1, :, :].astype(jnp.float32)
            rs_send_l[t, :, :] = val_l.astype(jnp.bfloat16)
            rdma_l = pltpu.make_async_remote_copy(
                src_ref=rs_send_l.at[t], dst_ref=rs_recv_l.at[t],
                send_sem=rs_ssem_l.at[t], recv_sem=rs_rsem_l.at[t],
                device_id=(left,), device_id_type=pl.DeviceIdType.MESH,
            )
            rdma_l.start()

            rdma_r.wait()
            rdma_l.wait()

        own_r = lax.rem(my + 1, N_DEV)
        q_r = (part_ref[pl.ds(own_r * ch, ch), :]
               + rs_recv_r[N_DEV - 2, :, :].astype(jnp.float32))
        ag_r[0, :, :] = q_r.astype(jnp.bfloat16)
        own_l = lax.rem(my + N_DEV - 1, N_DEV)
        q_l = (part_ref[pl.ds(half + own_l * ch, ch), :]
               + rs_recv_l[N_DEV - 2, :, :].astype(jnp.float32))
        ag_l[0, :, :] = q_l.astype(jnp.bfloat16)

        out_ref[pl.ds(own_r * ch, ch), :] = (
            out_ref[pl.ds(own_r * ch, ch), :] + q_r)
        out_ref[pl.ds(half + own_l * ch, ch), :] = (
            out_ref[pl.ds(half + own_l * ch, ch), :] + q_l)

        for t in range(N_DEV - 1):
            rdma_r = pltpu.make_async_remote_copy(
                src_ref=ag_r.at[t], dst_ref=ag_r.at[t + 1],
                send_sem=ag_ssem_r.at[t], recv_sem=ag_rsem_r.at[t],
                device_id=(right,), device_id_type=pl.DeviceIdType.MESH,
            )
            rdma_r.start()
            rdma_l = pltpu.make_async_remote_copy(
                src_ref=ag_l.at[t], dst_ref=ag_l.at[t + 1],
                send_sem=ag_ssem_l.at[t], recv_sem=ag_rsem_l.at[t],
                device_id=(left,), device_id_type=pl.DeviceIdType.MESH,
            )
            rdma_l.start()
            rdma_r.wait()
            rdma_l.wait()

            c_r = lax.rem(my - t + N_DEV, N_DEV)
            out_ref[pl.ds(c_r * ch, ch), :] = (
                out_ref[pl.ds(c_r * ch, ch), :]
                + ag_r[t + 1, :, :].astype(jnp.float32))
            c_l = lax.rem(my + t, N_DEV)
            out_ref[pl.ds(half + c_l * ch, ch), :] = (
                out_ref[pl.ds(half + c_l * ch, ch), :]
                + ag_l[t + 1, :, :].astype(jnp.float32))

    return pl.pallas_call(
        body,
        out_shape=jax.ShapeDtypeStruct((n_tok, d_ff), jnp.float32),
        in_specs=[pl.BlockSpec(memory_space=pltpu.VMEM)] * 5,
        out_specs=pl.BlockSpec(memory_space=pltpu.VMEM),
        scratch_shapes=[
            pltpu.VMEM((n_tok, d_ff), jnp.float32),
            pltpu.VMEM((N_DEV - 1, ch, d_ff), jnp.bfloat16),
            pltpu.VMEM((N_DEV - 1, ch, d_ff), jnp.bfloat16),
            pltpu.VMEM((N_DEV - 1, ch, d_ff), jnp.bfloat16),
            pltpu.VMEM((N_DEV - 1, ch, d_ff), jnp.bfloat16),
            pltpu.VMEM((N_DEV, ch, d_ff), jnp.bfloat16),
            pltpu.VMEM((N_DEV, ch, d_ff), jnp.bfloat16),
            pltpu.SemaphoreType.DMA((N_DEV - 1,)),
            pltpu.SemaphoreType.DMA((N_DEV - 1,)),
            pltpu.SemaphoreType.DMA((N_DEV - 1,)),
            pltpu.SemaphoreType.DMA((N_DEV - 1,)),
            pltpu.SemaphoreType.DMA((N_DEV - 1,)),
            pltpu.SemaphoreType.DMA((N_DEV - 1,)),
            pltpu.SemaphoreType.DMA((N_DEV - 1,)),
            pltpu.SemaphoreType.DMA((N_DEV - 1,)),
        ],
        compiler_params=pltpu.CompilerParams(collective_id=0),
    )(x, router_W, route_idx, expert_W, shared_W)


# device time: 43713 ns/iter; 2.1620x vs baseline; 1.1129x over previous
import jax
import jax.numpy as jnp
from jax import lax
from jax.experimental import pallas as pl
from jax.experimental.pallas import tpu as pltpu

N_DEV = 4


def kernel(x, router_W, route_idx, expert_W, shared_W):
    n_tok, d_model = x.shape
    e_loc, _, d_ff = expert_W.shape
    n_exp = router_W.shape[1]
    half = n_tok // 2
    ch = half // N_DEV

    def body(x_ref, rw_ref, idx_ref, ew_ref, sw_ref, out_ref,
             xs_ref,
             rs_send_r, rs_recv_r, rs_send_l, rs_recv_l,
             ag_r, ag_l,
             rs_ssem_r, rs_rsem_r, rs_ssem_l, rs_rsem_l,
             ag_ssem_r, ag_rsem_r, ag_ssem_l, ag_rsem_l):
        my = lax.axis_index("i")
        left = lax.rem(my + N_DEV - 1, N_DEV)
        right = lax.rem(my + 1, N_DEV)

        barrier_sem = pltpu.get_barrier_semaphore()
        for nbr in [left, right]:
            pl.semaphore_signal(
                barrier_sem, inc=1,
                device_id=(nbr,), device_id_type=pl.DeviceIdType.MESH,
            )

        xf = x_ref[:, :]
        scores = jnp.dot(xf, rw_ref[:, :], preferred_element_type=jnp.float32)
        probs = jax.nn.softmax(scores, axis=-1)
        idx = idx_ref[:, :]
        eye = lax.broadcasted_iota(jnp.int32, (n_tok, n_exp), 1)
        onehot = (idx == eye).astype(jnp.float32)
        p_sel = jnp.sum(probs * onehot, axis=-1, keepdims=True)

        for e in range(e_loc):
            e_glob = my * e_loc + e
            coef = jnp.where(idx == e_glob, p_sel, 0.0)
            xs_ref[e, :, :] = (xf * coef).astype(jnp.bfloat16)

        ew_b = [ew_ref[e].astype(jnp.bfloat16) for e in range(e_loc)]

        def pchunk(row):
            s = None
            for e in range(e_loc):
                m = jnp.dot(xs_ref[e, pl.ds(row, ch), :], ew_b[e],
                            preferred_element_type=jnp.float32)
                s = m if s is None else s + m
            return s

        cur_r = pchunk(my * ch)
        cur_l = pchunk(half + my * ch)

        pl.semaphore_wait(barrier_sem, 2)

        for t in range(N_DEV - 1):
            rs_send_r[t, :, :] = cur_r.astype(jnp.bfloat16)
            rdma_r = pltpu.make_async_remote_copy(
                src_ref=rs_send_r.at[t], dst_ref=rs_recv_r.at[t],
                send_sem=rs_ssem_r.at[t], recv_sem=rs_rsem_r.at[t],
                device_id=(right,), device_id_type=pl.DeviceIdType.MESH,
            )
            rdma_r.start()
            rs_send_l[t, :, :] = cur_l.astype(jnp.bfloat16)
            rdma_l = pltpu.make_async_remote_copy(
                src_ref=rs_send_l.at[t], dst_ref=rs_recv_l.at[t],
                send_sem=rs_ssem_l.at[t], recv_sem=rs_rsem_l.at[t],
                device_id=(left,), device_id_type=pl.DeviceIdType.MESH,
            )
            rdma_l.start()

            nxt_r = pchunk(lax.rem(my - t - 1 + N_DEV, N_DEV) * ch)
            nxt_l = pchunk(half + lax.rem(my + t + 1, N_DEV) * ch)

            rdma_r.wait()
            rdma_l.wait()
            cur_r = nxt_r + rs_recv_r[t, :, :].astype(jnp.float32)
            cur_l = nxt_l + rs_recv_l[t, :, :].astype(jnp.float32)

        ag_r[0, :, :] = cur_r.astype(jnp.bfloat16)
        ag_l[0, :, :] = cur_l.astype(jnp.bfloat16)

        xb = xf.astype(jnp.bfloat16)
        own_r = lax.rem(my + 1, N_DEV)
        own_l = lax.rem(my + N_DEV - 1, N_DEV)
        for t in range(N_DEV - 1):
            rdma_r = pltpu.make_async_remote_copy(
                src_ref=ag_r.at[t], dst_ref=ag_r.at[t + 1],
                send_sem=ag_ssem_r.at[t], recv_sem=ag_rsem_r.at[t],
                device_id=(right,), device_id_type=pl.DeviceIdType.MESH,
            )
            rdma_r.start()
            rdma_l = pltpu.make_async_remote_copy(
                src_ref=ag_l.at[t], dst_ref=ag_l.at[t + 1],
                send_sem=ag_ssem_l.at[t], recv_sem=ag_rsem_l.at[t],
                device_id=(left,), device_id_type=pl.DeviceIdType.MESH,
            )
            rdma_l.start()

            if t == 0:
                out_ref[:, :] = jnp.dot(
                    xb, sw_ref[:, :].astype(jnp.bfloat16),
                    preferred_element_type=jnp.float32,
                )
                out_ref[pl.ds(own_r * ch, ch), :] = (
                    out_ref[pl.ds(own_r * ch, ch), :] + cur_r)
                out_ref[pl.ds(half + own_l * ch, ch), :] = (
                    out_ref[pl.ds(half + own_l * ch, ch), :] + cur_l)
            else:
                c_r = lax.rem(my - t + 1 + N_DEV, N_DEV)
                out_ref[pl.ds(c_r * ch, ch), :] = (
                    out_ref[pl.ds(c_r * ch, ch), :]
                    + ag_r[t, :, :].astype(jnp.float32))
                c_l = lax.rem(my + t - 1, N_DEV)
                out_ref[pl.ds(half + c_l * ch, ch), :] = (
                    out_ref[pl.ds(half + c_l * ch, ch), :]
                    + ag_l[t, :, :].astype(jnp.float32))

            rdma_r.wait()
            rdma_l.wait()

        c_r = lax.rem(my - N_DEV + 2 + N_DEV, N_DEV)
        out_ref[pl.ds(c_r * ch, ch), :] = (
            out_ref[pl.ds(c_r * ch, ch), :]
            + ag_r[N_DEV - 1, :, :].astype(jnp.float32))
        c_l = lax.rem(my + N_DEV - 2, N_DEV)
        out_ref[pl.ds(half + c_l * ch, ch), :] = (
            out_ref[pl.ds(half + c_l * ch, ch), :]
            + ag_l[N_DEV - 1, :, :].astype(jnp.float32))

    return pl.pallas_call(
        body,
        out_shape=jax.ShapeDtypeStruct((n_tok, d_ff), jnp.float32),
        in_specs=[pl.BlockSpec(memory_space=pltpu.VMEM)] * 5,
        out_specs=pl.BlockSpec(memory_space=pltpu.VMEM),
        scratch_shapes=[
            pltpu.VMEM((e_loc, n_tok, d_model), jnp.bfloat16),
            pltpu.VMEM((N_DEV - 1, ch, d_ff), jnp.bfloat16),
            pltpu.VMEM((N_DEV - 1, ch, d_ff), jnp.bfloat16),
            pltpu.VMEM((N_DEV - 1, ch, d_ff), jnp.bfloat16),
            pltpu.VMEM((N_DEV - 1, ch, d_ff), jnp.bfloat16),
            pltpu.VMEM((N_DEV, ch, d_ff), jnp.bfloat16),
            pltpu.VMEM((N_DEV, ch, d_ff), jnp.bfloat16),
            pltpu.SemaphoreType.DMA((N_DEV - 1,)),
            pltpu.SemaphoreType.DMA((N_DEV - 1,)),
            pltpu.SemaphoreType.DMA((N_DEV - 1,)),
            pltpu.SemaphoreType.DMA((N_DEV - 1,)),
            pltpu.SemaphoreType.DMA((N_DEV - 1,)),
            pltpu.SemaphoreType.DMA((N_DEV - 1,)),
            pltpu.SemaphoreType.DMA((N_DEV - 1,)),
            pltpu.SemaphoreType.DMA((N_DEV - 1,)),
        ],
        compiler_params=pltpu.CompilerParams(collective_id=0),
    )(x, router_W, route_idx, expert_W, shared_W)
